# baseline (speedup 1.0000x reference)
import functools

import jax
import jax.numpy as jnp
from jax import lax
from jax.experimental import pallas as pl
from jax.experimental.pallas import tpu as pltpu
from jax.experimental.pallas import tpu_sc as plsc

N_NODES = 10000
N_EDGES = 320000
N_RES = 1000
NC, NS, LANES = 2, 16, 16
NW = NC * NS

_GCH = 2560
_NCHUNKS = N_EDGES // _GCH


def _sc_edge_gather(table_flat, src, dst):
    mesh = plsc.VectorSubcoreMesh(core_axis_name="c", subcore_axis_name="s")

    @functools.partial(
        pl.kernel,
        out_type=jax.ShapeDtypeStruct((16, N_EDGES), jnp.float32),
        mesh=mesh,
        scratch_types=[
            pltpu.VMEM((N_NODES * 8,), jnp.float32),
            pltpu.VMEM((_GCH,), jnp.int32),
            pltpu.VMEM((_GCH,), jnp.int32),
            pltpu.VMEM((16 * _GCH,), jnp.float32),
        ],
        compiler_params=pltpu.CompilerParams(needs_layout_passes=False),
    )
    def k(tab_hbm, src_hbm, dst_hbm, out_hbm, tab_v, si_v, di_v, gb_v):
        wid = lax.axis_index("s") * NC + lax.axis_index("c")
        pltpu.sync_copy(tab_hbm, tab_v)
        nch = jnp.where(wid < _NCHUNKS - NW * (_NCHUNKS // NW),
                        _NCHUNKS // NW + 1, _NCHUNKS // NW)

        def chunk(i, carry):
            base = (wid + i * NW) * _GCH
            pltpu.sync_copy(src_hbm.at[pl.ds(base, _GCH)], si_v)
            pltpu.sync_copy(dst_hbm.at[pl.ds(base, _GCH)], di_v)

            def grp(g, c2):
                s16 = si_v[pl.ds(g * 16, 16)] * 8
                d16 = di_v[pl.ds(g * 16, 16)] * 8
                for c in range(3):
                    gb_v[pl.ds(c * _GCH + g * 16, 16)] = plsc.load_gather(
                        tab_v, [s16 + c])
                    gb_v[pl.ds((3 + c) * _GCH + g * 16, 16)] = \
                        plsc.load_gather(tab_v, [d16 + c])
                    gb_v[pl.ds((6 + c) * _GCH + g * 16, 16)] = \
                        plsc.load_gather(tab_v, [s16 + 3 + c])
                    gb_v[pl.ds((9 + c) * _GCH + g * 16, 16)] = \
                        plsc.load_gather(tab_v, [d16 + 3 + c])
                return c2

            lax.fori_loop(0, _GCH // 16, grp, 0)
            for c in range(12):
                pltpu.sync_copy(gb_v.at[pl.ds(c * _GCH, _GCH)],
                                out_hbm.at[c, pl.ds(base, _GCH)])
            return carry

        lax.fori_loop(0, nch, chunk, 0)

    return k(table_flat, src, dst)


_NPAD = 10240
_SBLK = 2560
_SNCH = N_EDGES // _SBLK


def _vperm(x, i):
    return jnp.take_along_axis(x, i, axis=0)


def _sc_segmax(h2a, h2b, dsta, dstb):
    mesh = plsc.VectorSubcoreMesh(core_axis_name="c", subcore_axis_name="s")
    f32 = jnp.float32

    @functools.partial(
        pl.kernel,
        out_type=(jax.ShapeDtypeStruct((384, _NPAD), f32),
                  jax.ShapeDtypeStruct((384, _NPAD), f32)),
        mesh=mesh,
        scratch_types=[
            pltpu.VMEM((8 * _NPAD,), f32),
            pltpu.VMEM((8 * _SBLK,), f32),
            pltpu.VMEM((_SBLK,), jnp.int32),
        ],
        compiler_params=pltpu.CompilerParams(needs_layout_passes=False),
    )
    def k(h2a_hbm, h2b_hbm, da_hbm, db_hbm, oa_hbm, ob_hbm, acc_v, h_v, d_v):
        wid = lax.axis_index("s") * NC + lax.axis_index("c")
        iota = lax.iota(jnp.int32, 16)
        ninf = jnp.full((16,), -jnp.inf, f32)
        zeros = jnp.zeros((16,), f32)

        def run(h2_hbm, d_hbm, o_hbm, row0):
            def ini(i, c):
                acc_v[pl.ds(i * 16, 16)] = ninf
                return c
            lax.fori_loop(0, 8 * _NPAD // 16, ini, 0)

            def chunk(ch, c):
                base = ch * _SBLK
                pltpu.sync_copy(d_hbm.at[pl.ds(base, _SBLK)], d_v)
                for f in range(8):
                    pltpu.sync_copy(h2_hbm.at[row0 + f, pl.ds(base, _SBLK)],
                                    h_v.at[pl.ds(f * _SBLK, _SBLK)])

                def grp(g, c2):
                    d16 = d_v[pl.ds(g * 16, 16)]
                    ks, pm = plsc.sort_key_val(d16, iota)
                    kprev = _vperm(ks, jnp.maximum(iota - 1, 0))
                    dup = (ks == kprev) & (iota >= 1)
                    dup_any = jnp.max(jnp.where(dup, 1, 0))
                    vals = [h_v[pl.ds(f * _SBLK + g * 16, 16)]
                            for f in range(8)]

                    def fast():
                        for f in range(8):
                            idx = d16 + f * _NPAD
                            cur = plsc.load_gather(acc_v, [idx])
                            plsc.store_scatter(acc_v, [idx],
                                               jnp.maximum(cur, vals[f]))

                    def slow():
                        knext = _vperm(ks, jnp.minimum(iota + 1, 15))
                        mlast = (ks != knext) | (iota == 15)
                        sames = []
                        for kk in (1, 2, 4, 8):
                            kp = _vperm(ks, jnp.maximum(iota - kk, 0))
                            sames.append((ks == kp) & (iota >= kk))
                        for f in range(8):
                            vp = _vperm(vals[f], pm)
                            for kk, sm in zip((1, 2, 4, 8), sames):
                                sh = _vperm(vp, jnp.maximum(iota - kk, 0))
                                vp = jnp.where(sm, jnp.maximum(vp, sh), vp)
                            idx = ks + f * _NPAD
                            cur = plsc.load_gather(acc_v, [idx], mask=mlast)
                            plsc.store_scatter(acc_v, [idx],
                                               jnp.maximum(cur, vp),
                                               mask=mlast)

                    lax.cond(dup_any > 0, slow, fast)
                    return c2

                lax.fori_loop(0, _SBLK // 16, grp, 0)
                return c

            lax.fori_loop(0, _SNCH, chunk, 0)

            def fin(i, c):
                v = acc_v[pl.ds(i * 16, 16)]
                acc_v[pl.ds(i * 16, 16)] = jnp.where(v == ninf, zeros, v)
                return c
            lax.fori_loop(0, 8 * _NPAD // 16, fin, 0)
            for f in range(8):
                pltpu.sync_copy(acc_v.at[pl.ds(f * _NPAD, _NPAD)],
                                o_hbm.at[row0 + f, :])

        for k3 in range(3):
            gid = wid + NW * k3
            lax.cond(
                gid >= 48,
                lambda gid=gid: run(h2b_hbm, db_hbm, ob_hbm, (gid - 48) * 8),
                lambda gid=gid: run(h2a_hbm, da_hbm, oa_hbm, gid * 8),
            )

    return k(h2a, h2b, dsta, dstb)


_EBLK = 2560


def _edge_mlp_kernel(g_ref, w1s_ref, b1_ref, lw1_ref, lb1_ref, w2s_ref,
                     b2_ref, lw2_ref, lb2_ref, o_ref):
    g = g_ref[...]
    d = g[0:3, :] - g[3:6, :]
    nj = g[6:9, :]
    ni = g[9:12, :]
    dn = jnp.sqrt(jnp.sum(d * d, axis=0, keepdims=True) + 1e-12)

    def ang(v1, v2):
        cx = v1[1:2, :] * v2[2:3, :] - v1[2:3, :] * v2[1:2, :]
        cy = v1[2:3, :] * v2[0:1, :] - v1[0:1, :] * v2[2:3, :]
        cz = v1[0:1, :] * v2[1:2, :] - v1[1:2, :] * v2[0:1, :]
        cn = jnp.sqrt(cx * cx + cy * cy + cz * cz + 1e-12)
        dt = jnp.sum(v1 * v2, axis=0, keepdims=True)
        return jnp.arctan2(cn, dt)

    ppf = jnp.concatenate([dn, ang(ni, d), ang(nj, d), ang(ni, nj)], axis=0)
    ones4 = jnp.ones((1, 4), jnp.float32)
    ones128 = jnp.ones((1, 128), jnp.float32)
    for i in range(3):
        w1t = w1s_ref[4 * i:4 * i + 4, :]
        p = jax.nn.relu(
            lax.dot_general(w1t, ppf, (((1,), (0,)), ((), ())),
                            preferred_element_type=jnp.float32)
            + b1_ref[4 * i:4 * i + 4, :])
        m = lax.dot_general(ones4, p, (((1,), (0,)), ((), ())),
                            preferred_element_type=jnp.float32) * 0.25
        pc = p - m
        v = lax.dot_general(ones4, pc * pc, (((1,), (0,)), ((), ())),
                            preferred_element_type=jnp.float32) * 0.25
        h1 = pc / jnp.sqrt(v + 1e-5) * lw1_ref[4 * i:4 * i + 4, :] \
            + lb1_ref[4 * i:4 * i + 4, :]
        w2t = w2s_ref[128 * i:128 * i + 128, :]
        q = jax.nn.relu(
            lax.dot_general(w2t, h1, (((1,), (0,)), ((), ())),
                            preferred_element_type=jnp.float32)
            + b2_ref[128 * i:128 * i + 128, :])
        m2 = lax.dot_general(ones128, q, (((1,), (0,)), ((), ())),
                             preferred_element_type=jnp.float32) * (1.0 / 128.0)
        qc = q - m2
        v2 = lax.dot_general(ones128, qc * qc, (((1,), (0,)), ((), ())),
                             preferred_element_type=jnp.float32) * (1.0 / 128.0)
        o_ref[128 * i:128 * i + 128, :] = (
            qc / jnp.sqrt(v2 + 1e-5) * lw2_ref[128 * i:128 * i + 128, :]
            + lb2_ref[128 * i:128 * i + 128, :])


def _edge_mlp(g, w1s, b1c, lw1c, lb1c, w2s, b2c, lw2c, lb2c):
    grid = (N_EDGES // _EBLK,)
    wspec = lambda r: pl.BlockSpec((r, 1), lambda i: (0, 0))
    return pl.pallas_call(
        _edge_mlp_kernel,
        grid=grid,
        in_specs=[
            pl.BlockSpec((16, _EBLK), lambda i: (0, i)),
            pl.BlockSpec((12, 4), lambda i: (0, 0)),
            wspec(12), wspec(12), wspec(12),
            pl.BlockSpec((384, 4), lambda i: (0, 0)),
            wspec(384), wspec(384), wspec(384),
        ],
        out_specs=pl.BlockSpec((384, _EBLK), lambda i: (0, i)),
        out_shape=jax.ShapeDtypeStruct((384, N_EDGES), jnp.float32),
    )(g, w1s, b1c, lw1c, lb1c, w2s, b2c, lw2c, lb2c)


def _mlp_ln_kernel(x_ref, w_ref, b_ref, lw_ref, lb_ref, o_ref):
    h = jax.nn.relu(
        lax.dot_general(x_ref[...], w_ref[...], (((1,), (0,)), ((), ())),
                        preferred_element_type=jnp.float32) + b_ref[...])
    m = jnp.mean(h, axis=-1, keepdims=True)
    v = jnp.mean((h - m) ** 2, axis=-1, keepdims=True)
    o_ref[...] = (h - m) / jnp.sqrt(v + 1e-5) * lw_ref[...] + lb_ref[...]


def _mlp_ln(x, W, b, lw, lb, blk):
    n, din = x.shape
    dout = W.shape[1]
    return pl.pallas_call(
        _mlp_ln_kernel,
        grid=(n // blk,),
        in_specs=[
            pl.BlockSpec((blk, din), lambda i: (i, 0)),
            pl.BlockSpec((din, dout), lambda i: (0, 0)),
            pl.BlockSpec((dout,), lambda i: (0,)),
            pl.BlockSpec((dout,), lambda i: (0,)),
            pl.BlockSpec((dout,), lambda i: (0,)),
        ],
        out_specs=pl.BlockSpec((blk, dout), lambda i: (i, 0)),
        out_shape=jax.ShapeDtypeStruct((n, dout), jnp.float32),
    )(x, W, b, lw, lb)


def kernel(pos_A, normal_A, pos_B, normal_B, conv_W1, conv_b1, conv_ln1_w,
           conv_ln1_b, conv_W2, conv_b2, conv_ln2_w, conv_ln2_b, Wa, ba,
           lna_w, lna_b, Wr, br, lnr_w, lnr_b, Wl, bl, edge_index_A,
           edge_index_B, residue_ids_A, residue_ids_B, src_res_idx,
           tgt_res_idx):
    f32 = jnp.float32
    w1s = jnp.transpose(conv_W1, (0, 2, 1)).reshape(12, 4)
    b1c = conv_b1.reshape(12, 1)
    lw1c = conv_ln1_w.reshape(12, 1)
    lb1c = conv_ln1_b.reshape(12, 1)
    w2s = jnp.transpose(conv_W2, (0, 2, 1)).reshape(384, 4)
    b2c = conv_b2.reshape(384, 1)
    lw2c = conv_ln2_w.reshape(384, 1)
    lb2c = conv_ln2_b.reshape(384, 1)

    def edge_stage(pos, normal, edge_index):
        table = jnp.concatenate(
            [pos, normal, jnp.zeros((N_NODES, 2), f32)], axis=1).reshape(-1)
        g = _sc_edge_gather(table, edge_index[0], edge_index[1])
        return _edge_mlp(g, w1s, b1c, lw1c, lb1c, w2s, b2c, lw2c, lb2c)

    h2tA = edge_stage(pos_A, normal_A, edge_index_A)
    h2tB = edge_stage(pos_B, normal_B, edge_index_B)
    nfA, nfB = _sc_segmax(h2tA, h2tB, edge_index_A[1], edge_index_B[1])

    def node_stage(nf, res_ids):
        atom = _mlp_ln(nf[:, :N_NODES].T, Wa, ba, lna_w, lna_b, blk=1000)
        res = jax.ops.segment_max(atom, res_ids, num_segments=N_RES)
        res = jnp.where(jnp.isneginf(res), 0.0, res)
        return _mlp_ln(res, Wr, br, lnr_w, lnr_b, blk=1000)

    res_A = node_stage(nfA, residue_ids_A)
    res_B = node_stage(nfB, residue_ids_B)
    x_s = res_A[src_res_idx]
    x_t = res_B[tgt_res_idx]
    out = jax.nn.sigmoid((x_s * x_t) @ Wl + bl)[:, 0]
    return out


# speedup vs baseline: 1.3621x; 1.3621x over previous
import functools

import jax
import jax.numpy as jnp
from jax import lax
from jax.experimental import pallas as pl
from jax.experimental.pallas import tpu as pltpu
from jax.experimental.pallas import tpu_sc as plsc

N_NODES = 10000
N_EDGES = 320000
N_RES = 1000
NC, NS, LANES = 2, 16, 16
NW = NC * NS

_GCH = 2560
_NCHUNKS = N_EDGES // _GCH


def _sc_edge_gather(table_flat, src, dst):
    mesh = plsc.VectorSubcoreMesh(core_axis_name="c", subcore_axis_name="s")

    @functools.partial(
        pl.kernel,
        out_type=jax.ShapeDtypeStruct((16, N_EDGES), jnp.float32),
        mesh=mesh,
        scratch_types=[
            pltpu.VMEM((N_NODES * 8,), jnp.float32),
            pltpu.VMEM((_GCH,), jnp.int32),
            pltpu.VMEM((_GCH,), jnp.int32),
            pltpu.VMEM((16 * _GCH,), jnp.float32),
        ],
        compiler_params=pltpu.CompilerParams(needs_layout_passes=False),
    )
    def k(tab_hbm, src_hbm, dst_hbm, out_hbm, tab_v, si_v, di_v, gb_v):
        wid = lax.axis_index("s") * NC + lax.axis_index("c")
        pltpu.sync_copy(tab_hbm, tab_v)
        nch = jnp.where(wid < _NCHUNKS - NW * (_NCHUNKS // NW),
                        _NCHUNKS // NW + 1, _NCHUNKS // NW)

        def chunk(i, carry):
            base = (wid + i * NW) * _GCH
            pltpu.sync_copy(src_hbm.at[pl.ds(base, _GCH)], si_v)
            pltpu.sync_copy(dst_hbm.at[pl.ds(base, _GCH)], di_v)

            def grp(g, c2):
                s16 = si_v[pl.ds(g * 16, 16)] * 8
                d16 = di_v[pl.ds(g * 16, 16)] * 8
                for c in range(3):
                    gb_v[pl.ds(c * _GCH + g * 16, 16)] = plsc.load_gather(
                        tab_v, [s16 + c])
                    gb_v[pl.ds((3 + c) * _GCH + g * 16, 16)] = \
                        plsc.load_gather(tab_v, [d16 + c])
                    gb_v[pl.ds((6 + c) * _GCH + g * 16, 16)] = \
                        plsc.load_gather(tab_v, [s16 + 3 + c])
                    gb_v[pl.ds((9 + c) * _GCH + g * 16, 16)] = \
                        plsc.load_gather(tab_v, [d16 + 3 + c])
                return c2

            lax.fori_loop(0, _GCH // 16, grp, 0)
            for c in range(12):
                pltpu.sync_copy(gb_v.at[pl.ds(c * _GCH, _GCH)],
                                out_hbm.at[c, pl.ds(base, _GCH)])
            return carry

        lax.fori_loop(0, nch, chunk, 0)

    return k(table_flat, src, dst)


_NPAD = 10240
_SBLK = 2560
_SNCH = N_EDGES // _SBLK


def _vperm(x, i):
    return jnp.take_along_axis(x, i, axis=0)


def _sc_segmax(h2a, h2b, dsta, dstb):
    mesh = plsc.VectorSubcoreMesh(core_axis_name="c", subcore_axis_name="s")
    f32 = jnp.float32

    @functools.partial(
        pl.kernel,
        out_type=(jax.ShapeDtypeStruct((384, _NPAD), f32),
                  jax.ShapeDtypeStruct((384, _NPAD), f32)),
        mesh=mesh,
        scratch_types=[
            pltpu.VMEM((8 * _NPAD,), f32),
            pltpu.VMEM((2 * 8 * _SBLK,), f32),
            pltpu.VMEM((2 * _SBLK,), jnp.int32),
            pltpu.SemaphoreType.DMA,
            pltpu.SemaphoreType.DMA,
        ],
        compiler_params=pltpu.CompilerParams(needs_layout_passes=False),
    )
    def k(h2a_hbm, h2b_hbm, da_hbm, db_hbm, oa_hbm, ob_hbm, acc_v, h_v, d_v,
          sem0, sem1):
        wid = lax.axis_index("s") * NC + lax.axis_index("c")
        iota = lax.iota(jnp.int32, 16)
        ninf = jnp.full((16,), -jnp.inf, f32)
        zeros = jnp.zeros((16,), f32)

        def run(h2_hbm, d_hbm, o_hbm, row0):
            def ini(i, c):
                acc_v[pl.ds(i * 16, 16)] = ninf
                return c
            lax.fori_loop(0, 8 * _NPAD // 16, ini, 0)

            def fire(ch, b, sem):
                base = ch * _SBLK
                pltpu.async_copy(d_hbm.at[pl.ds(base, _SBLK)],
                                 d_v.at[pl.ds(b * _SBLK, _SBLK)], sem)
                for f in range(8):
                    pltpu.async_copy(
                        h2_hbm.at[row0 + f, pl.ds(base, _SBLK)],
                        h_v.at[pl.ds((b * 8 + f) * _SBLK, _SBLK)], sem)

            def drain(ch, b, sem):
                base = ch * _SBLK
                pltpu.make_async_copy(
                    d_hbm.at[pl.ds(base, _SBLK)],
                    d_v.at[pl.ds(b * _SBLK, _SBLK)], sem).wait()
                for f in range(8):
                    pltpu.make_async_copy(
                        h2_hbm.at[row0 + f, pl.ds(base, _SBLK)],
                        h_v.at[pl.ds((b * 8 + f) * _SBLK, _SBLK)], sem).wait()

            def process(b):
                def grp(g, c2):
                    d16 = d_v[pl.ds(b * _SBLK + g * 16, 16)]
                    ks, pm = plsc.sort_key_val(d16, iota)
                    kprev = _vperm(ks, jnp.maximum(iota - 1, 0))
                    dup = (ks == kprev) & (iota >= 1)
                    dup_any = jnp.max(jnp.where(dup, 1, 0))
                    vals = [h_v[pl.ds((b * 8 + f) * _SBLK + g * 16, 16)]
                            for f in range(8)]

                    def fast():
                        for f in range(8):
                            idx = d16 + f * _NPAD
                            cur = plsc.load_gather(acc_v, [idx])
                            plsc.store_scatter(acc_v, [idx],
                                               jnp.maximum(cur, vals[f]))

                    def slow():
                        knext = _vperm(ks, jnp.minimum(iota + 1, 15))
                        mlast = (ks != knext) | (iota == 15)
                        sames = []
                        for kk in (1, 2, 4, 8):
                            kp = _vperm(ks, jnp.maximum(iota - kk, 0))
                            sames.append((ks == kp) & (iota >= kk))
                        for f in range(8):
                            vp = _vperm(vals[f], pm)
                            for kk, sm in zip((1, 2, 4, 8), sames):
                                sh = _vperm(vp, jnp.maximum(iota - kk, 0))
                                vp = jnp.where(sm, jnp.maximum(vp, sh), vp)
                            idx = ks + f * _NPAD
                            cur = plsc.load_gather(acc_v, [idx], mask=mlast)
                            plsc.store_scatter(acc_v, [idx],
                                               jnp.maximum(cur, vp),
                                               mask=mlast)

                    lax.cond(dup_any > 0, slow, fast)
                    return c2

                lax.fori_loop(0, _SBLK // 16, grp, 0)

            fire(0, 0, sem0)

            def dbl(j, c):
                fire(2 * j + 1, 1, sem1)
                drain(2 * j, 0, sem0)
                process(0)
                fire(2 * j + 2, 0, sem0)
                drain(2 * j + 1, 1, sem1)
                process(1)
                return c

            lax.fori_loop(0, (_SNCH - 1) // 2, dbl, 0)
            drain(_SNCH - 1, 0, sem0)
            process(0)

            def fin(i, c):
                v = acc_v[pl.ds(i * 16, 16)]
                acc_v[pl.ds(i * 16, 16)] = jnp.where(v == ninf, zeros, v)
                return c
            lax.fori_loop(0, 8 * _NPAD // 16, fin, 0)
            for f in range(8):
                pltpu.sync_copy(acc_v.at[pl.ds(f * _NPAD, _NPAD)],
                                o_hbm.at[row0 + f, :])

        for k3 in range(3):
            gid = wid + NW * k3
            lax.cond(
                gid >= 48,
                lambda gid=gid: run(h2b_hbm, db_hbm, ob_hbm, (gid - 48) * 8),
                lambda gid=gid: run(h2a_hbm, da_hbm, oa_hbm, gid * 8),
            )

    return k(h2a, h2b, dsta, dstb)


_EBLK = 2560


def _edge_mlp_kernel(g_ref, w1s_ref, b1_ref, lw1_ref, lb1_ref, w2s_ref,
                     b2_ref, lw2_ref, lb2_ref, o_ref):
    g = g_ref[...]
    d = g[0:3, :] - g[3:6, :]
    nj = g[6:9, :]
    ni = g[9:12, :]
    dn = jnp.sqrt(jnp.sum(d * d, axis=0, keepdims=True) + 1e-12)

    def ang(v1, v2):
        cx = v1[1:2, :] * v2[2:3, :] - v1[2:3, :] * v2[1:2, :]
        cy = v1[2:3, :] * v2[0:1, :] - v1[0:1, :] * v2[2:3, :]
        cz = v1[0:1, :] * v2[1:2, :] - v1[1:2, :] * v2[0:1, :]
        cn = jnp.sqrt(cx * cx + cy * cy + cz * cz + 1e-12)
        dt = jnp.sum(v1 * v2, axis=0, keepdims=True)
        return jnp.arctan2(cn, dt)

    ppf = jnp.concatenate([dn, ang(ni, d), ang(nj, d), ang(ni, nj)], axis=0)
    ones4 = jnp.ones((1, 4), jnp.float32)
    ones128 = jnp.ones((1, 128), jnp.float32)
    for i in range(3):
        w1t = w1s_ref[4 * i:4 * i + 4, :]
        p = jax.nn.relu(
            lax.dot_general(w1t, ppf, (((1,), (0,)), ((), ())),
                            preferred_element_type=jnp.float32)
            + b1_ref[4 * i:4 * i + 4, :])
        m = lax.dot_general(ones4, p, (((1,), (0,)), ((), ())),
                            preferred_element_type=jnp.float32) * 0.25
        pc = p - m
        v = lax.dot_general(ones4, pc * pc, (((1,), (0,)), ((), ())),
                            preferred_element_type=jnp.float32) * 0.25
        h1 = pc / jnp.sqrt(v + 1e-5) * lw1_ref[4 * i:4 * i + 4, :] \
            + lb1_ref[4 * i:4 * i + 4, :]
        w2t = w2s_ref[128 * i:128 * i + 128, :]
        q = jax.nn.relu(
            lax.dot_general(w2t, h1, (((1,), (0,)), ((), ())),
                            preferred_element_type=jnp.float32)
            + b2_ref[128 * i:128 * i + 128, :])
        m2 = lax.dot_general(ones128, q, (((1,), (0,)), ((), ())),
                             preferred_element_type=jnp.float32) * (1.0 / 128.0)
        qc = q - m2
        v2 = lax.dot_general(ones128, qc * qc, (((1,), (0,)), ((), ())),
                             preferred_element_type=jnp.float32) * (1.0 / 128.0)
        o_ref[128 * i:128 * i + 128, :] = (
            qc / jnp.sqrt(v2 + 1e-5) * lw2_ref[128 * i:128 * i + 128, :]
            + lb2_ref[128 * i:128 * i + 128, :])


def _edge_mlp(g, w1s, b1c, lw1c, lb1c, w2s, b2c, lw2c, lb2c):
    grid = (N_EDGES // _EBLK,)
    wspec = lambda r: pl.BlockSpec((r, 1), lambda i: (0, 0))
    return pl.pallas_call(
        _edge_mlp_kernel,
        grid=grid,
        in_specs=[
            pl.BlockSpec((16, _EBLK), lambda i: (0, i)),
            pl.BlockSpec((12, 4), lambda i: (0, 0)),
            wspec(12), wspec(12), wspec(12),
            pl.BlockSpec((384, 4), lambda i: (0, 0)),
            wspec(384), wspec(384), wspec(384),
        ],
        out_specs=pl.BlockSpec((384, _EBLK), lambda i: (0, i)),
        out_shape=jax.ShapeDtypeStruct((384, N_EDGES), jnp.float32),
    )(g, w1s, b1c, lw1c, lb1c, w2s, b2c, lw2c, lb2c)


def _mlp_ln_kernel(x_ref, w_ref, b_ref, lw_ref, lb_ref, o_ref):
    h = jax.nn.relu(
        lax.dot_general(x_ref[...], w_ref[...], (((1,), (0,)), ((), ())),
                        preferred_element_type=jnp.float32) + b_ref[...])
    m = jnp.mean(h, axis=-1, keepdims=True)
    v = jnp.mean((h - m) ** 2, axis=-1, keepdims=True)
    o_ref[...] = (h - m) / jnp.sqrt(v + 1e-5) * lw_ref[...] + lb_ref[...]


def _mlp_ln(x, W, b, lw, lb, blk):
    n, din = x.shape
    dout = W.shape[1]
    return pl.pallas_call(
        _mlp_ln_kernel,
        grid=(n // blk,),
        in_specs=[
            pl.BlockSpec((blk, din), lambda i: (i, 0)),
            pl.BlockSpec((din, dout), lambda i: (0, 0)),
            pl.BlockSpec((dout,), lambda i: (0,)),
            pl.BlockSpec((dout,), lambda i: (0,)),
            pl.BlockSpec((dout,), lambda i: (0,)),
        ],
        out_specs=pl.BlockSpec((blk, dout), lambda i: (i, 0)),
        out_shape=jax.ShapeDtypeStruct((n, dout), jnp.float32),
    )(x, W, b, lw, lb)


def kernel(pos_A, normal_A, pos_B, normal_B, conv_W1, conv_b1, conv_ln1_w,
           conv_ln1_b, conv_W2, conv_b2, conv_ln2_w, conv_ln2_b, Wa, ba,
           lna_w, lna_b, Wr, br, lnr_w, lnr_b, Wl, bl, edge_index_A,
           edge_index_B, residue_ids_A, residue_ids_B, src_res_idx,
           tgt_res_idx):
    f32 = jnp.float32
    w1s = jnp.transpose(conv_W1, (0, 2, 1)).reshape(12, 4)
    b1c = conv_b1.reshape(12, 1)
    lw1c = conv_ln1_w.reshape(12, 1)
    lb1c = conv_ln1_b.reshape(12, 1)
    w2s = jnp.transpose(conv_W2, (0, 2, 1)).reshape(384, 4)
    b2c = conv_b2.reshape(384, 1)
    lw2c = conv_ln2_w.reshape(384, 1)
    lb2c = conv_ln2_b.reshape(384, 1)

    def edge_stage(pos, normal, edge_index):
        table = jnp.concatenate(
            [pos, normal, jnp.zeros((N_NODES, 2), f32)], axis=1).reshape(-1)
        g = _sc_edge_gather(table, edge_index[0], edge_index[1])
        return _edge_mlp(g, w1s, b1c, lw1c, lb1c, w2s, b2c, lw2c, lb2c)

    h2tA = edge_stage(pos_A, normal_A, edge_index_A)
    h2tB = edge_stage(pos_B, normal_B, edge_index_B)
    nfA, nfB = _sc_segmax(h2tA, h2tB, edge_index_A[1], edge_index_B[1])

    def node_stage(nf, res_ids):
        atom = _mlp_ln(nf[:, :N_NODES].T, Wa, ba, lna_w, lna_b, blk=1000)
        res = jax.ops.segment_max(atom, res_ids, num_segments=N_RES)
        res = jnp.where(jnp.isneginf(res), 0.0, res)
        return _mlp_ln(res, Wr, br, lnr_w, lnr_b, blk=1000)

    res_A = node_stage(nfA, residue_ids_A)
    res_B = node_stage(nfB, residue_ids_B)
    x_s = res_A[src_res_idx]
    x_t = res_B[tgt_res_idx]
    out = jax.nn.sigmoid((x_s * x_t) @ Wl + bl)[:, 0]
    return out


# speedup vs baseline: 1.4173x; 1.0405x over previous
import functools

import jax
import jax.numpy as jnp
from jax import lax
from jax.experimental import pallas as pl
from jax.experimental.pallas import tpu as pltpu
from jax.experimental.pallas import tpu_sc as plsc

N_NODES = 10000
N_EDGES = 320000
N_RES = 1000
NC, NS, LANES = 2, 16, 16
NW = NC * NS

_GCH = 2560
_NCHUNKS = N_EDGES // _GCH


def _sc_edge_gather(table_flat, src, dst):
    mesh = plsc.VectorSubcoreMesh(core_axis_name="c", subcore_axis_name="s")

    @functools.partial(
        pl.kernel,
        out_type=jax.ShapeDtypeStruct((16, N_EDGES), jnp.float32),
        mesh=mesh,
        scratch_types=[
            pltpu.VMEM((N_NODES * 8,), jnp.float32),
            pltpu.VMEM((_GCH,), jnp.int32),
            pltpu.VMEM((_GCH,), jnp.int32),
            pltpu.VMEM((16 * _GCH,), jnp.float32),
        ],
        compiler_params=pltpu.CompilerParams(needs_layout_passes=False),
    )
    def k(tab_hbm, src_hbm, dst_hbm, out_hbm, tab_v, si_v, di_v, gb_v):
        wid = lax.axis_index("s") * NC + lax.axis_index("c")
        pltpu.sync_copy(tab_hbm, tab_v)
        nch = jnp.where(wid < _NCHUNKS - NW * (_NCHUNKS // NW),
                        _NCHUNKS // NW + 1, _NCHUNKS // NW)

        def chunk(i, carry):
            base = (wid + i * NW) * _GCH
            pltpu.sync_copy(src_hbm.at[pl.ds(base, _GCH)], si_v)
            pltpu.sync_copy(dst_hbm.at[pl.ds(base, _GCH)], di_v)

            def grp(g, c2):
                s16 = si_v[pl.ds(g * 16, 16)] * 8
                d16 = di_v[pl.ds(g * 16, 16)] * 8
                for c in range(3):
                    gb_v[pl.ds(c * _GCH + g * 16, 16)] = plsc.load_gather(
                        tab_v, [s16 + c])
                    gb_v[pl.ds((3 + c) * _GCH + g * 16, 16)] = \
                        plsc.load_gather(tab_v, [d16 + c])
                    gb_v[pl.ds((6 + c) * _GCH + g * 16, 16)] = \
                        plsc.load_gather(tab_v, [s16 + 3 + c])
                    gb_v[pl.ds((9 + c) * _GCH + g * 16, 16)] = \
                        plsc.load_gather(tab_v, [d16 + 3 + c])
                return c2

            lax.fori_loop(0, _GCH // 16, grp, 0)
            for c in range(12):
                pltpu.sync_copy(gb_v.at[pl.ds(c * _GCH, _GCH)],
                                out_hbm.at[c, pl.ds(base, _GCH)])
            return carry

        lax.fori_loop(0, nch, chunk, 0)

    return k(table_flat, src, dst)


_NPAD = 10240
_SBLK = 2560
_SNCH = N_EDGES // _SBLK


def _vperm(x, i):
    return jnp.take_along_axis(x, i, axis=0)


def _sc_segmax(h2a, h2b, dsta, dstb):
    mesh = plsc.VectorSubcoreMesh(core_axis_name="c", subcore_axis_name="s")
    f32 = jnp.float32

    @functools.partial(
        pl.kernel,
        out_type=(jax.ShapeDtypeStruct((384, _NPAD), f32),
                  jax.ShapeDtypeStruct((384, _NPAD), f32)),
        mesh=mesh,
        scratch_types=[
            pltpu.VMEM((8 * _NPAD,), f32),
            pltpu.VMEM((2 * 8 * _SBLK,), f32),
            pltpu.VMEM((2 * _SBLK,), jnp.int32),
            pltpu.VMEM((2048,), jnp.int32),
            pltpu.SemaphoreType.DMA,
            pltpu.SemaphoreType.DMA,
        ],
        compiler_params=pltpu.CompilerParams(needs_layout_passes=False),
    )
    def k(h2a_hbm, h2b_hbm, da_hbm, db_hbm, oa_hbm, ob_hbm, acc_v, h_v, d_v,
          tag_v, sem0, sem1):
        wid = lax.axis_index("s") * NC + lax.axis_index("c")
        iota = lax.iota(jnp.int32, 16)
        ninf = jnp.full((16,), -jnp.inf, f32)
        zeros = jnp.zeros((16,), f32)

        def run(h2_hbm, d_hbm, o_hbm, row0):
            def ini(i, c):
                acc_v[pl.ds(i * 16, 16)] = ninf
                return c
            lax.fori_loop(0, 8 * _NPAD // 16, ini, 0)

            def fire(ch, b, sem):
                base = ch * _SBLK
                pltpu.async_copy(d_hbm.at[pl.ds(base, _SBLK)],
                                 d_v.at[pl.ds(b * _SBLK, _SBLK)], sem)
                for f in range(8):
                    pltpu.async_copy(
                        h2_hbm.at[row0 + f, pl.ds(base, _SBLK)],
                        h_v.at[pl.ds((b * 8 + f) * _SBLK, _SBLK)], sem)

            def drain(ch, b, sem):
                base = ch * _SBLK
                pltpu.make_async_copy(
                    d_hbm.at[pl.ds(base, _SBLK)],
                    d_v.at[pl.ds(b * _SBLK, _SBLK)], sem).wait()
                for f in range(8):
                    pltpu.make_async_copy(
                        h2_hbm.at[row0 + f, pl.ds(base, _SBLK)],
                        h_v.at[pl.ds((b * 8 + f) * _SBLK, _SBLK)], sem).wait()

            def process(b):
                def grp(g, c2):
                    d16 = d_v[pl.ds(b * _SBLK + g * 16, 16)]
                    dh = d16 & 2047
                    plsc.store_scatter(tag_v, [dh], iota)
                    rb = plsc.load_gather(tag_v, [dh])
                    dup_any = jnp.any(rb != iota)
                    vals = [h_v[pl.ds((b * 8 + f) * _SBLK + g * 16, 16)]
                            for f in range(8)]

                    def fast():
                        for f in range(8):
                            idx = d16 + f * _NPAD
                            cur = plsc.load_gather(acc_v, [idx])
                            plsc.store_scatter(acc_v, [idx],
                                               jnp.maximum(cur, vals[f]))

                    def slow():
                        ks, pm = plsc.sort_key_val(d16, iota)
                        knext = _vperm(ks, jnp.minimum(iota + 1, 15))
                        mlast = (ks != knext) | (iota == 15)
                        sames = []
                        for kk in (1, 2, 4, 8):
                            kp = _vperm(ks, jnp.maximum(iota - kk, 0))
                            sames.append((ks == kp) & (iota >= kk))
                        for f in range(8):
                            vp = _vperm(vals[f], pm)
                            for kk, sm in zip((1, 2, 4, 8), sames):
                                sh = _vperm(vp, jnp.maximum(iota - kk, 0))
                                vp = jnp.where(sm, jnp.maximum(vp, sh), vp)
                            idx = ks + f * _NPAD
                            cur = plsc.load_gather(acc_v, [idx], mask=mlast)
                            plsc.store_scatter(acc_v, [idx],
                                               jnp.maximum(cur, vp),
                                               mask=mlast)

                    lax.cond(dup_any, slow, fast)
                    return c2

                lax.fori_loop(0, _SBLK // 16, grp, 0)

            fire(0, 0, sem0)

            def dbl(j, c):
                fire(2 * j + 1, 1, sem1)
                drain(2 * j, 0, sem0)
                process(0)
                fire(2 * j + 2, 0, sem0)
                drain(2 * j + 1, 1, sem1)
                process(1)
                return c

            lax.fori_loop(0, (_SNCH - 1) // 2, dbl, 0)
            drain(_SNCH - 1, 0, sem0)
            process(0)

            def fin(i, c):
                v = acc_v[pl.ds(i * 16, 16)]
                acc_v[pl.ds(i * 16, 16)] = jnp.where(v == ninf, zeros, v)
                return c
            lax.fori_loop(0, 8 * _NPAD // 16, fin, 0)
            for f in range(8):
                pltpu.sync_copy(acc_v.at[pl.ds(f * _NPAD, _NPAD)],
                                o_hbm.at[row0 + f, :])

        for k3 in range(3):
            gid = wid + NW * k3
            lax.cond(
                gid >= 48,
                lambda gid=gid: run(h2b_hbm, db_hbm, ob_hbm, (gid - 48) * 8),
                lambda gid=gid: run(h2a_hbm, da_hbm, oa_hbm, gid * 8),
            )

    return k(h2a, h2b, dsta, dstb)


_EBLK = 2560


def _edge_mlp_kernel(g_ref, w1s_ref, b1_ref, lw1_ref, lb1_ref, w2s_ref,
                     b2_ref, lw2_ref, lb2_ref, o_ref):
    g = g_ref[...]
    d = g[0:3, :] - g[3:6, :]
    nj = g[6:9, :]
    ni = g[9:12, :]
    dn = jnp.sqrt(jnp.sum(d * d, axis=0, keepdims=True) + 1e-12)

    def ang(v1, v2):
        cx = v1[1:2, :] * v2[2:3, :] - v1[2:3, :] * v2[1:2, :]
        cy = v1[2:3, :] * v2[0:1, :] - v1[0:1, :] * v2[2:3, :]
        cz = v1[0:1, :] * v2[1:2, :] - v1[1:2, :] * v2[0:1, :]
        cn = jnp.sqrt(cx * cx + cy * cy + cz * cz + 1e-12)
        dt = jnp.sum(v1 * v2, axis=0, keepdims=True)
        return jnp.arctan2(cn, dt)

    ppf = jnp.concatenate([dn, ang(ni, d), ang(nj, d), ang(ni, nj)], axis=0)
    ones4 = jnp.ones((1, 4), jnp.float32)
    ones128 = jnp.ones((1, 128), jnp.float32)
    for i in range(3):
        w1t = w1s_ref[4 * i:4 * i + 4, :]
        p = jax.nn.relu(
            lax.dot_general(w1t, ppf, (((1,), (0,)), ((), ())),
                            preferred_element_type=jnp.float32)
            + b1_ref[4 * i:4 * i + 4, :])
        m = lax.dot_general(ones4, p, (((1,), (0,)), ((), ())),
                            preferred_element_type=jnp.float32) * 0.25
        pc = p - m
        v = lax.dot_general(ones4, pc * pc, (((1,), (0,)), ((), ())),
                            preferred_element_type=jnp.float32) * 0.25
        h1 = pc / jnp.sqrt(v + 1e-5) * lw1_ref[4 * i:4 * i + 4, :] \
            + lb1_ref[4 * i:4 * i + 4, :]
        w2t = w2s_ref[128 * i:128 * i + 128, :]
        q = jax.nn.relu(
            lax.dot_general(w2t, h1, (((1,), (0,)), ((), ())),
                            preferred_element_type=jnp.float32)
            + b2_ref[128 * i:128 * i + 128, :])
        m2 = lax.dot_general(ones128, q, (((1,), (0,)), ((), ())),
                             preferred_element_type=jnp.float32) * (1.0 / 128.0)
        qc = q - m2
        v2 = lax.dot_general(ones128, qc * qc, (((1,), (0,)), ((), ())),
                             preferred_element_type=jnp.float32) * (1.0 / 128.0)
        o_ref[128 * i:128 * i + 128, :] = (
            qc / jnp.sqrt(v2 + 1e-5) * lw2_ref[128 * i:128 * i + 128, :]
            + lb2_ref[128 * i:128 * i + 128, :])


def _edge_mlp(g, w1s, b1c, lw1c, lb1c, w2s, b2c, lw2c, lb2c):
    grid = (N_EDGES // _EBLK,)
    wspec = lambda r: pl.BlockSpec((r, 1), lambda i: (0, 0))
    return pl.pallas_call(
        _edge_mlp_kernel,
        grid=grid,
        in_specs=[
            pl.BlockSpec((16, _EBLK), lambda i: (0, i)),
            pl.BlockSpec((12, 4), lambda i: (0, 0)),
            wspec(12), wspec(12), wspec(12),
            pl.BlockSpec((384, 4), lambda i: (0, 0)),
            wspec(384), wspec(384), wspec(384),
        ],
        out_specs=pl.BlockSpec((384, _EBLK), lambda i: (0, i)),
        out_shape=jax.ShapeDtypeStruct((384, N_EDGES), jnp.float32),
    )(g, w1s, b1c, lw1c, lb1c, w2s, b2c, lw2c, lb2c)


def _mlp_ln_kernel(x_ref, w_ref, b_ref, lw_ref, lb_ref, o_ref):
    h = jax.nn.relu(
        lax.dot_general(x_ref[...], w_ref[...], (((1,), (0,)), ((), ())),
                        preferred_element_type=jnp.float32) + b_ref[...])
    m = jnp.mean(h, axis=-1, keepdims=True)
    v = jnp.mean((h - m) ** 2, axis=-1, keepdims=True)
    o_ref[...] = (h - m) / jnp.sqrt(v + 1e-5) * lw_ref[...] + lb_ref[...]


def _mlp_ln(x, W, b, lw, lb, blk):
    n, din = x.shape
    dout = W.shape[1]
    return pl.pallas_call(
        _mlp_ln_kernel,
        grid=(n // blk,),
        in_specs=[
            pl.BlockSpec((blk, din), lambda i: (i, 0)),
            pl.BlockSpec((din, dout), lambda i: (0, 0)),
            pl.BlockSpec((dout,), lambda i: (0,)),
            pl.BlockSpec((dout,), lambda i: (0,)),
            pl.BlockSpec((dout,), lambda i: (0,)),
        ],
        out_specs=pl.BlockSpec((blk, dout), lambda i: (i, 0)),
        out_shape=jax.ShapeDtypeStruct((n, dout), jnp.float32),
    )(x, W, b, lw, lb)


def kernel(pos_A, normal_A, pos_B, normal_B, conv_W1, conv_b1, conv_ln1_w,
           conv_ln1_b, conv_W2, conv_b2, conv_ln2_w, conv_ln2_b, Wa, ba,
           lna_w, lna_b, Wr, br, lnr_w, lnr_b, Wl, bl, edge_index_A,
           edge_index_B, residue_ids_A, residue_ids_B, src_res_idx,
           tgt_res_idx):
    f32 = jnp.float32
    w1s = jnp.transpose(conv_W1, (0, 2, 1)).reshape(12, 4)
    b1c = conv_b1.reshape(12, 1)
    lw1c = conv_ln1_w.reshape(12, 1)
    lb1c = conv_ln1_b.reshape(12, 1)
    w2s = jnp.transpose(conv_W2, (0, 2, 1)).reshape(384, 4)
    b2c = conv_b2.reshape(384, 1)
    lw2c = conv_ln2_w.reshape(384, 1)
    lb2c = conv_ln2_b.reshape(384, 1)

    def edge_stage(pos, normal, edge_index):
        table = jnp.concatenate(
            [pos, normal, jnp.zeros((N_NODES, 2), f32)], axis=1).reshape(-1)
        g = _sc_edge_gather(table, edge_index[0], edge_index[1])
        return _edge_mlp(g, w1s, b1c, lw1c, lb1c, w2s, b2c, lw2c, lb2c)

    h2tA = edge_stage(pos_A, normal_A, edge_index_A)
    h2tB = edge_stage(pos_B, normal_B, edge_index_B)
    nfA, nfB = _sc_segmax(h2tA, h2tB, edge_index_A[1], edge_index_B[1])

    def node_stage(nf, res_ids):
        atom = _mlp_ln(nf[:, :N_NODES].T, Wa, ba, lna_w, lna_b, blk=1000)
        res = jax.ops.segment_max(atom, res_ids, num_segments=N_RES)
        res = jnp.where(jnp.isneginf(res), 0.0, res)
        return _mlp_ln(res, Wr, br, lnr_w, lnr_b, blk=1000)

    res_A = node_stage(nfA, residue_ids_A)
    res_B = node_stage(nfB, residue_ids_B)
    x_s = res_A[src_res_idx]
    x_t = res_B[tgt_res_idx]
    out = jax.nn.sigmoid((x_s * x_t) @ Wl + bl)[:, 0]
    return out


# speedup vs baseline: 1.8001x; 1.2701x over previous
import functools

import jax
import jax.numpy as jnp
from jax import lax
from jax.experimental import pallas as pl
from jax.experimental.pallas import tpu as pltpu
from jax.experimental.pallas import tpu_sc as plsc

N_NODES = 10000
N_EDGES = 320000
N_RES = 1000
NC, NS, LANES = 2, 16, 16
NW = NC * NS

_GCH = 2560
_NCHUNKS = N_EDGES // _GCH


def _sc_edge_gather(table_flat, src, dst):
    mesh = plsc.VectorSubcoreMesh(core_axis_name="c", subcore_axis_name="s")

    @functools.partial(
        pl.kernel,
        out_type=jax.ShapeDtypeStruct((16, N_EDGES), jnp.float32),
        mesh=mesh,
        scratch_types=[
            pltpu.VMEM((N_NODES * 8,), jnp.float32),
            pltpu.VMEM((_GCH,), jnp.int32),
            pltpu.VMEM((_GCH,), jnp.int32),
            pltpu.VMEM((16 * _GCH,), jnp.float32),
        ],
        compiler_params=pltpu.CompilerParams(needs_layout_passes=False),
    )
    def k(tab_hbm, src_hbm, dst_hbm, out_hbm, tab_v, si_v, di_v, gb_v):
        wid = lax.axis_index("s") * NC + lax.axis_index("c")
        pltpu.sync_copy(tab_hbm, tab_v)
        nch = jnp.where(wid < _NCHUNKS - NW * (_NCHUNKS // NW),
                        _NCHUNKS // NW + 1, _NCHUNKS // NW)

        def chunk(i, carry):
            base = (wid + i * NW) * _GCH
            pltpu.sync_copy(src_hbm.at[pl.ds(base, _GCH)], si_v)
            pltpu.sync_copy(dst_hbm.at[pl.ds(base, _GCH)], di_v)

            def grp(g, c2):
                s16 = si_v[pl.ds(g * 16, 16)] * 8
                d16 = di_v[pl.ds(g * 16, 16)] * 8
                for c in range(3):
                    gb_v[pl.ds(c * _GCH + g * 16, 16)] = plsc.load_gather(
                        tab_v, [s16 + c])
                    gb_v[pl.ds((3 + c) * _GCH + g * 16, 16)] = \
                        plsc.load_gather(tab_v, [d16 + c])
                    gb_v[pl.ds((6 + c) * _GCH + g * 16, 16)] = \
                        plsc.load_gather(tab_v, [s16 + 3 + c])
                    gb_v[pl.ds((9 + c) * _GCH + g * 16, 16)] = \
                        plsc.load_gather(tab_v, [d16 + 3 + c])
                return c2

            lax.fori_loop(0, _GCH // 16, grp, 0)
            for c in range(12):
                pltpu.sync_copy(gb_v.at[pl.ds(c * _GCH, _GCH)],
                                out_hbm.at[c, pl.ds(base, _GCH)])
            return carry

        lax.fori_loop(0, nch, chunk, 0)

    return k(table_flat, src, dst)


_NPAD = 10240
_SBLK = 2560
_SNCH = N_EDGES // _SBLK


def _vperm(x, i):
    return jnp.take_along_axis(x, i, axis=0)


def _sc_segmax(h2a, h2b, dsta, dstb):
    mesh = plsc.VectorSubcoreMesh(core_axis_name="c", subcore_axis_name="s")
    f32 = jnp.float32

    @functools.partial(
        pl.kernel,
        out_type=(jax.ShapeDtypeStruct((384, _NPAD), f32),
                  jax.ShapeDtypeStruct((384, _NPAD), f32)),
        mesh=mesh,
        scratch_types=[
            pltpu.VMEM((8 * _NPAD,), f32),
            pltpu.VMEM((2 * 8 * _SBLK,), f32),
            pltpu.VMEM((2 * _SBLK,), jnp.int32),
            pltpu.VMEM((2048,), jnp.int32),
            pltpu.SemaphoreType.DMA,
            pltpu.SemaphoreType.DMA,
        ],
        compiler_params=pltpu.CompilerParams(needs_layout_passes=False),
    )
    def k(h2a_hbm, h2b_hbm, da_hbm, db_hbm, oa_hbm, ob_hbm, acc_v, h_v, d_v,
          tag_v, sem0, sem1):
        wid = lax.axis_index("s") * NC + lax.axis_index("c")
        iota = lax.iota(jnp.int32, 16)
        ninf = jnp.full((16,), -jnp.inf, f32)
        zeros = jnp.zeros((16,), f32)

        def run(h2_hbm, d_hbm, o_hbm, row0):
            def ini(i, c):
                acc_v[pl.ds(i * 16, 16)] = ninf
                return c
            lax.fori_loop(0, 8 * _NPAD // 16, ini, 0)

            def fire(ch, b, sem):
                base = ch * _SBLK
                pltpu.async_copy(d_hbm.at[pl.ds(base, _SBLK)],
                                 d_v.at[pl.ds(b * _SBLK, _SBLK)], sem)
                for f in range(8):
                    pltpu.async_copy(
                        h2_hbm.at[row0 + f, pl.ds(base, _SBLK)],
                        h_v.at[pl.ds((b * 8 + f) * _SBLK, _SBLK)], sem)

            def drain(ch, b, sem):
                base = ch * _SBLK
                pltpu.make_async_copy(
                    d_hbm.at[pl.ds(base, _SBLK)],
                    d_v.at[pl.ds(b * _SBLK, _SBLK)], sem).wait()
                for f in range(8):
                    pltpu.make_async_copy(
                        h2_hbm.at[row0 + f, pl.ds(base, _SBLK)],
                        h_v.at[pl.ds((b * 8 + f) * _SBLK, _SBLK)], sem).wait()

            def process(b):
                def grp(g, c2):
                    d16 = d_v[pl.ds(b * _SBLK + g * 16, 16)]
                    dh = d16 & 2047
                    plsc.store_scatter(tag_v, [dh], iota)
                    rb = plsc.load_gather(tag_v, [dh])
                    dup_any = jnp.any(rb != iota)
                    vals = [h_v[pl.ds((b * 8 + f) * _SBLK + g * 16, 16)]
                            for f in range(8)]

                    def fast():
                        idxs = [d16 + f * _NPAD for f in range(8)]
                        curs = [plsc.load_gather(acc_v, [idxs[f]])
                                for f in range(8)]
                        for f in range(8):
                            plsc.store_scatter(acc_v, [idxs[f]],
                                               jnp.maximum(curs[f], vals[f]))

                    def slow():
                        ks, pm = plsc.sort_key_val(d16, iota)
                        knext = _vperm(ks, jnp.minimum(iota + 1, 15))
                        mlast = (ks != knext) | (iota == 15)
                        sames = []
                        for kk in (1, 2, 4, 8):
                            kp = _vperm(ks, jnp.maximum(iota - kk, 0))
                            sames.append((ks == kp) & (iota >= kk))
                        vps = []
                        for f in range(8):
                            vp = _vperm(vals[f], pm)
                            for kk, sm in zip((1, 2, 4, 8), sames):
                                sh = _vperm(vp, jnp.maximum(iota - kk, 0))
                                vp = jnp.where(sm, jnp.maximum(vp, sh), vp)
                            vps.append(vp)
                        idxs = [ks + f * _NPAD for f in range(8)]
                        curs = [plsc.load_gather(acc_v, [idxs[f]], mask=mlast)
                                for f in range(8)]
                        for f in range(8):
                            plsc.store_scatter(acc_v, [idxs[f]],
                                               jnp.maximum(curs[f], vps[f]),
                                               mask=mlast)

                    lax.cond(dup_any, slow, fast)
                    return c2

                lax.fori_loop(0, _SBLK // 16, grp, 0)

            fire(0, 0, sem0)

            def dbl(j, c):
                fire(2 * j + 1, 1, sem1)
                drain(2 * j, 0, sem0)
                process(0)
                fire(2 * j + 2, 0, sem0)
                drain(2 * j + 1, 1, sem1)
                process(1)
                return c

            lax.fori_loop(0, (_SNCH - 1) // 2, dbl, 0)
            drain(_SNCH - 1, 0, sem0)
            process(0)

            def fin(i, c):
                v = acc_v[pl.ds(i * 16, 16)]
                acc_v[pl.ds(i * 16, 16)] = jnp.where(v == ninf, zeros, v)
                return c
            lax.fori_loop(0, 8 * _NPAD // 16, fin, 0)
            for f in range(8):
                pltpu.sync_copy(acc_v.at[pl.ds(f * _NPAD, _NPAD)],
                                o_hbm.at[row0 + f, :])

        for k3 in range(3):
            gid = wid + NW * k3
            lax.cond(
                gid >= 48,
                lambda gid=gid: run(h2b_hbm, db_hbm, ob_hbm, (gid - 48) * 8),
                lambda gid=gid: run(h2a_hbm, da_hbm, oa_hbm, gid * 8),
            )

    return k(h2a, h2b, dsta, dstb)


_EBLK = 2560


def _edge_mlp_kernel(g_ref, w1s_ref, b1_ref, lw1_ref, lb1_ref, w2s_ref,
                     b2_ref, lw2_ref, lb2_ref, o_ref):
    g = g_ref[...]
    d = g[0:3, :] - g[3:6, :]
    nj = g[6:9, :]
    ni = g[9:12, :]
    dn = jnp.sqrt(jnp.sum(d * d, axis=0, keepdims=True) + 1e-12)

    def ang(v1, v2):
        cx = v1[1:2, :] * v2[2:3, :] - v1[2:3, :] * v2[1:2, :]
        cy = v1[2:3, :] * v2[0:1, :] - v1[0:1, :] * v2[2:3, :]
        cz = v1[0:1, :] * v2[1:2, :] - v1[1:2, :] * v2[0:1, :]
        cn = jnp.sqrt(cx * cx + cy * cy + cz * cz + 1e-12)
        dt = jnp.sum(v1 * v2, axis=0, keepdims=True)
        return jnp.arctan2(cn, dt)

    ppf = jnp.concatenate([dn, ang(ni, d), ang(nj, d), ang(ni, nj)], axis=0)
    ones4 = jnp.ones((1, 4), jnp.float32)
    ones128 = jnp.ones((1, 128), jnp.float32)
    for i in range(3):
        w1t = w1s_ref[4 * i:4 * i + 4, :]
        p = jax.nn.relu(
            lax.dot_general(w1t, ppf, (((1,), (0,)), ((), ())),
                            preferred_element_type=jnp.float32)
            + b1_ref[4 * i:4 * i + 4, :])
        m = lax.dot_general(ones4, p, (((1,), (0,)), ((), ())),
                            preferred_element_type=jnp.float32) * 0.25
        pc = p - m
        v = lax.dot_general(ones4, pc * pc, (((1,), (0,)), ((), ())),
                            preferred_element_type=jnp.float32) * 0.25
        h1 = pc / jnp.sqrt(v + 1e-5) * lw1_ref[4 * i:4 * i + 4, :] \
            + lb1_ref[4 * i:4 * i + 4, :]
        w2t = w2s_ref[128 * i:128 * i + 128, :]
        q = jax.nn.relu(
            lax.dot_general(w2t, h1, (((1,), (0,)), ((), ())),
                            preferred_element_type=jnp.float32)
            + b2_ref[128 * i:128 * i + 128, :])
        m2 = lax.dot_general(ones128, q, (((1,), (0,)), ((), ())),
                             preferred_element_type=jnp.float32) * (1.0 / 128.0)
        qc = q - m2
        v2 = lax.dot_general(ones128, qc * qc, (((1,), (0,)), ((), ())),
                             preferred_element_type=jnp.float32) * (1.0 / 128.0)
        o_ref[128 * i:128 * i + 128, :] = (
            qc / jnp.sqrt(v2 + 1e-5) * lw2_ref[128 * i:128 * i + 128, :]
            + lb2_ref[128 * i:128 * i + 128, :])


def _edge_mlp(g, w1s, b1c, lw1c, lb1c, w2s, b2c, lw2c, lb2c):
    grid = (N_EDGES // _EBLK,)
    wspec = lambda r: pl.BlockSpec((r, 1), lambda i: (0, 0))
    return pl.pallas_call(
        _edge_mlp_kernel,
        grid=grid,
        in_specs=[
            pl.BlockSpec((16, _EBLK), lambda i: (0, i)),
            pl.BlockSpec((12, 4), lambda i: (0, 0)),
            wspec(12), wspec(12), wspec(12),
            pl.BlockSpec((384, 4), lambda i: (0, 0)),
            wspec(384), wspec(384), wspec(384),
        ],
        out_specs=pl.BlockSpec((384, _EBLK), lambda i: (0, i)),
        out_shape=jax.ShapeDtypeStruct((384, N_EDGES), jnp.float32),
    )(g, w1s, b1c, lw1c, lb1c, w2s, b2c, lw2c, lb2c)


def _mlp_ln_kernel(x_ref, w_ref, b_ref, lw_ref, lb_ref, o_ref):
    h = jax.nn.relu(
        lax.dot_general(x_ref[...], w_ref[...], (((1,), (0,)), ((), ())),
                        preferred_element_type=jnp.float32) + b_ref[...])
    m = jnp.mean(h, axis=-1, keepdims=True)
    v = jnp.mean((h - m) ** 2, axis=-1, keepdims=True)
    o_ref[...] = (h - m) / jnp.sqrt(v + 1e-5) * lw_ref[...] + lb_ref[...]


def _mlp_ln(x, W, b, lw, lb, blk):
    n, din = x.shape
    dout = W.shape[1]
    return pl.pallas_call(
        _mlp_ln_kernel,
        grid=(n // blk,),
        in_specs=[
            pl.BlockSpec((blk, din), lambda i: (i, 0)),
            pl.BlockSpec((din, dout), lambda i: (0, 0)),
            pl.BlockSpec((dout,), lambda i: (0,)),
            pl.BlockSpec((dout,), lambda i: (0,)),
            pl.BlockSpec((dout,), lambda i: (0,)),
        ],
        out_specs=pl.BlockSpec((blk, dout), lambda i: (i, 0)),
        out_shape=jax.ShapeDtypeStruct((n, dout), jnp.float32),
    )(x, W, b, lw, lb)


def kernel(pos_A, normal_A, pos_B, normal_B, conv_W1, conv_b1, conv_ln1_w,
           conv_ln1_b, conv_W2, conv_b2, conv_ln2_w, conv_ln2_b, Wa, ba,
           lna_w, lna_b, Wr, br, lnr_w, lnr_b, Wl, bl, edge_index_A,
           edge_index_B, residue_ids_A, residue_ids_B, src_res_idx,
           tgt_res_idx):
    f32 = jnp.float32
    w1s = jnp.transpose(conv_W1, (0, 2, 1)).reshape(12, 4)
    b1c = conv_b1.reshape(12, 1)
    lw1c = conv_ln1_w.reshape(12, 1)
    lb1c = conv_ln1_b.reshape(12, 1)
    w2s = jnp.transpose(conv_W2, (0, 2, 1)).reshape(384, 4)
    b2c = conv_b2.reshape(384, 1)
    lw2c = conv_ln2_w.reshape(384, 1)
    lb2c = conv_ln2_b.reshape(384, 1)

    def edge_stage(pos, normal, edge_index):
        table = jnp.concatenate(
            [pos, normal, jnp.zeros((N_NODES, 2), f32)], axis=1).reshape(-1)
        g = _sc_edge_gather(table, edge_index[0], edge_index[1])
        return _edge_mlp(g, w1s, b1c, lw1c, lb1c, w2s, b2c, lw2c, lb2c)

    h2tA = edge_stage(pos_A, normal_A, edge_index_A)
    h2tB = edge_stage(pos_B, normal_B, edge_index_B)
    nfA, nfB = _sc_segmax(h2tA, h2tB, edge_index_A[1], edge_index_B[1])

    def node_stage(nf, res_ids):
        atom = _mlp_ln(nf[:, :N_NODES].T, Wa, ba, lna_w, lna_b, blk=1000)
        res = jax.ops.segment_max(atom, res_ids, num_segments=N_RES)
        res = jnp.where(jnp.isneginf(res), 0.0, res)
        return _mlp_ln(res, Wr, br, lnr_w, lnr_b, blk=1000)

    res_A = node_stage(nfA, residue_ids_A)
    res_B = node_stage(nfB, residue_ids_B)
    x_s = res_A[src_res_idx]
    x_t = res_B[tgt_res_idx]
    out = jax.nn.sigmoid((x_s * x_t) @ Wl + bl)[:, 0]
    return out


# speedup vs baseline: 1.8560x; 1.0311x over previous
import functools

import jax
import jax.numpy as jnp
from jax import lax
from jax.experimental import pallas as pl
from jax.experimental.pallas import tpu as pltpu
from jax.experimental.pallas import tpu_sc as plsc

N_NODES = 10000
N_EDGES = 320000
N_RES = 1000
NC, NS, LANES = 2, 16, 16
NW = NC * NS

_GCH = 2560
_NCHUNKS = N_EDGES // _GCH


def _sc_edge_gather(table_flat, src, dst):
    mesh = plsc.VectorSubcoreMesh(core_axis_name="c", subcore_axis_name="s")

    @functools.partial(
        pl.kernel,
        out_type=jax.ShapeDtypeStruct((16, N_EDGES), jnp.float32),
        mesh=mesh,
        scratch_types=[
            pltpu.VMEM((N_NODES * 8,), jnp.float32),
            pltpu.VMEM((_GCH,), jnp.int32),
            pltpu.VMEM((_GCH,), jnp.int32),
            pltpu.VMEM((16 * _GCH,), jnp.float32),
        ],
        compiler_params=pltpu.CompilerParams(needs_layout_passes=False),
    )
    def k(tab_hbm, src_hbm, dst_hbm, out_hbm, tab_v, si_v, di_v, gb_v):
        wid = lax.axis_index("s") * NC + lax.axis_index("c")
        pltpu.sync_copy(tab_hbm, tab_v)
        nch = jnp.where(wid < _NCHUNKS - NW * (_NCHUNKS // NW),
                        _NCHUNKS // NW + 1, _NCHUNKS // NW)

        def chunk(i, carry):
            base = (wid + i * NW) * _GCH
            pltpu.sync_copy(src_hbm.at[pl.ds(base, _GCH)], si_v)
            pltpu.sync_copy(dst_hbm.at[pl.ds(base, _GCH)], di_v)

            def grp(g, c2):
                s16 = si_v[pl.ds(g * 16, 16)] * 8
                d16 = di_v[pl.ds(g * 16, 16)] * 8
                for c in range(3):
                    gb_v[pl.ds(c * _GCH + g * 16, 16)] = plsc.load_gather(
                        tab_v, [s16 + c])
                    gb_v[pl.ds((3 + c) * _GCH + g * 16, 16)] = \
                        plsc.load_gather(tab_v, [d16 + c])
                    gb_v[pl.ds((6 + c) * _GCH + g * 16, 16)] = \
                        plsc.load_gather(tab_v, [s16 + 3 + c])
                    gb_v[pl.ds((9 + c) * _GCH + g * 16, 16)] = \
                        plsc.load_gather(tab_v, [d16 + 3 + c])
                return c2

            lax.fori_loop(0, _GCH // 16, grp, 0)
            for c in range(12):
                pltpu.sync_copy(gb_v.at[pl.ds(c * _GCH, _GCH)],
                                out_hbm.at[c, pl.ds(base, _GCH)])
            return carry

        lax.fori_loop(0, nch, chunk, 0)

    return k(table_flat, src, dst)


_NPAD = 10240
_SBLK = 2560
_SNCH = N_EDGES // _SBLK


def _vperm(x, i):
    return jnp.take_along_axis(x, i, axis=0)


def _sc_segmax(h2a, h2b, dsta, dstb):
    mesh = plsc.VectorSubcoreMesh(core_axis_name="c", subcore_axis_name="s")
    f32 = jnp.float32

    @functools.partial(
        pl.kernel,
        out_type=(jax.ShapeDtypeStruct((384, _NPAD), f32),
                  jax.ShapeDtypeStruct((384, _NPAD), f32)),
        mesh=mesh,
        scratch_types=[
            [pltpu.VMEM((_NPAD,), f32) for _ in range(8)],
            pltpu.VMEM((2 * 8 * _SBLK,), f32),
            pltpu.VMEM((2 * _SBLK,), jnp.int32),
            pltpu.VMEM((2048,), jnp.int32),
            pltpu.SemaphoreType.DMA,
            pltpu.SemaphoreType.DMA,
        ],
        compiler_params=pltpu.CompilerParams(needs_layout_passes=False),
    )
    def k(h2a_hbm, h2b_hbm, da_hbm, db_hbm, oa_hbm, ob_hbm, accs, h_v, d_v,
          tag_v, sem0, sem1):
        wid = lax.axis_index("s") * NC + lax.axis_index("c")
        iota = lax.iota(jnp.int32, 16)
        ninf = jnp.full((16,), -jnp.inf, f32)
        zeros = jnp.zeros((16,), f32)

        def run(h2_hbm, d_hbm, o_hbm, row0):
            def ini(i, c):
                for f in range(8):
                    accs[f][pl.ds(i * 16, 16)] = ninf
                return c
            lax.fori_loop(0, _NPAD // 16, ini, 0)

            def fire(ch, b, sem):
                base = ch * _SBLK
                pltpu.async_copy(d_hbm.at[pl.ds(base, _SBLK)],
                                 d_v.at[pl.ds(b * _SBLK, _SBLK)], sem)
                for f in range(8):
                    pltpu.async_copy(
                        h2_hbm.at[row0 + f, pl.ds(base, _SBLK)],
                        h_v.at[pl.ds((b * 8 + f) * _SBLK, _SBLK)], sem)

            def drain(ch, b, sem):
                base = ch * _SBLK
                pltpu.make_async_copy(
                    d_hbm.at[pl.ds(base, _SBLK)],
                    d_v.at[pl.ds(b * _SBLK, _SBLK)], sem).wait()
                for f in range(8):
                    pltpu.make_async_copy(
                        h2_hbm.at[row0 + f, pl.ds(base, _SBLK)],
                        h_v.at[pl.ds((b * 8 + f) * _SBLK, _SBLK)], sem).wait()

            def process(b):
                def grp(g, c2):
                    d16 = d_v[pl.ds(b * _SBLK + g * 16, 16)]
                    dh = d16 & 2047
                    plsc.store_scatter(tag_v, [dh], iota)
                    rb = plsc.load_gather(tag_v, [dh])
                    dup_any = jnp.any(rb != iota)
                    vals = [h_v[pl.ds((b * 8 + f) * _SBLK + g * 16, 16)]
                            for f in range(8)]

                    def fast():
                        curs = [plsc.load_gather(accs[f], [d16])
                                for f in range(8)]
                        for f in range(8):
                            plsc.store_scatter(accs[f], [d16],
                                               jnp.maximum(curs[f], vals[f]))

                    def slow():
                        ks, pm = plsc.sort_key_val(d16, iota)
                        knext = _vperm(ks, jnp.minimum(iota + 1, 15))
                        mlast = (ks != knext) | (iota == 15)
                        sames = []
                        for kk in (1, 2, 4, 8):
                            kp = _vperm(ks, jnp.maximum(iota - kk, 0))
                            sames.append((ks == kp) & (iota >= kk))
                        vps = []
                        for f in range(8):
                            vp = _vperm(vals[f], pm)
                            for kk, sm in zip((1, 2, 4, 8), sames):
                                sh = _vperm(vp, jnp.maximum(iota - kk, 0))
                                vp = jnp.where(sm, jnp.maximum(vp, sh), vp)
                            vps.append(vp)
                        curs = [plsc.load_gather(accs[f], [ks], mask=mlast)
                                for f in range(8)]
                        for f in range(8):
                            plsc.store_scatter(accs[f], [ks],
                                               jnp.maximum(curs[f], vps[f]),
                                               mask=mlast)

                    lax.cond(dup_any, slow, fast)
                    return c2

                lax.fori_loop(0, _SBLK // 16, grp, 0)

            fire(0, 0, sem0)

            def dbl(j, c):
                fire(2 * j + 1, 1, sem1)
                drain(2 * j, 0, sem0)
                process(0)
                fire(2 * j + 2, 0, sem0)
                drain(2 * j + 1, 1, sem1)
                process(1)
                return c

            lax.fori_loop(0, (_SNCH - 1) // 2, dbl, 0)
            drain(_SNCH - 1, 0, sem0)
            process(0)

            def fin(i, c):
                for f in range(8):
                    v = accs[f][pl.ds(i * 16, 16)]
                    accs[f][pl.ds(i * 16, 16)] = jnp.where(v == ninf, zeros, v)
                return c
            lax.fori_loop(0, _NPAD // 16, fin, 0)
            for f in range(8):
                pltpu.sync_copy(accs[f], o_hbm.at[row0 + f, :])

        for k3 in range(3):
            gid = wid + NW * k3
            lax.cond(
                gid >= 48,
                lambda gid=gid: run(h2b_hbm, db_hbm, ob_hbm, (gid - 48) * 8),
                lambda gid=gid: run(h2a_hbm, da_hbm, oa_hbm, gid * 8),
            )

    return k(h2a, h2b, dsta, dstb)


_EBLK = 2560


def _edge_mlp_kernel(g_ref, w1s_ref, b1_ref, lw1_ref, lb1_ref, w2s_ref,
                     b2_ref, lw2_ref, lb2_ref, o_ref):
    g = g_ref[...]
    d = g[0:3, :] - g[3:6, :]
    nj = g[6:9, :]
    ni = g[9:12, :]
    dn = jnp.sqrt(jnp.sum(d * d, axis=0, keepdims=True) + 1e-12)

    def ang(v1, v2):
        cx = v1[1:2, :] * v2[2:3, :] - v1[2:3, :] * v2[1:2, :]
        cy = v1[2:3, :] * v2[0:1, :] - v1[0:1, :] * v2[2:3, :]
        cz = v1[0:1, :] * v2[1:2, :] - v1[1:2, :] * v2[0:1, :]
        cn = jnp.sqrt(cx * cx + cy * cy + cz * cz + 1e-12)
        dt = jnp.sum(v1 * v2, axis=0, keepdims=True)
        return jnp.arctan2(cn, dt)

    ppf = jnp.concatenate([dn, ang(ni, d), ang(nj, d), ang(ni, nj)], axis=0)
    ones4 = jnp.ones((1, 4), jnp.float32)
    ones128 = jnp.ones((1, 128), jnp.float32)
    for i in range(3):
        w1t = w1s_ref[4 * i:4 * i + 4, :]
        p = jax.nn.relu(
            lax.dot_general(w1t, ppf, (((1,), (0,)), ((), ())),
                            preferred_element_type=jnp.float32)
            + b1_ref[4 * i:4 * i + 4, :])
        m = lax.dot_general(ones4, p, (((1,), (0,)), ((), ())),
                            preferred_element_type=jnp.float32) * 0.25
        pc = p - m
        v = lax.dot_general(ones4, pc * pc, (((1,), (0,)), ((), ())),
                            preferred_element_type=jnp.float32) * 0.25
        h1 = pc / jnp.sqrt(v + 1e-5) * lw1_ref[4 * i:4 * i + 4, :] \
            + lb1_ref[4 * i:4 * i + 4, :]
        w2t = w2s_ref[128 * i:128 * i + 128, :]
        q = jax.nn.relu(
            lax.dot_general(w2t, h1, (((1,), (0,)), ((), ())),
                            preferred_element_type=jnp.float32)
            + b2_ref[128 * i:128 * i + 128, :])
        m2 = lax.dot_general(ones128, q, (((1,), (0,)), ((), ())),
                             preferred_element_type=jnp.float32) * (1.0 / 128.0)
        qc = q - m2
        v2 = lax.dot_general(ones128, qc * qc, (((1,), (0,)), ((), ())),
                             preferred_element_type=jnp.float32) * (1.0 / 128.0)
        o_ref[128 * i:128 * i + 128, :] = (
            qc / jnp.sqrt(v2 + 1e-5) * lw2_ref[128 * i:128 * i + 128, :]
            + lb2_ref[128 * i:128 * i + 128, :])


def _edge_mlp(g, w1s, b1c, lw1c, lb1c, w2s, b2c, lw2c, lb2c):
    grid = (N_EDGES // _EBLK,)
    wspec = lambda r: pl.BlockSpec((r, 1), lambda i: (0, 0))
    return pl.pallas_call(
        _edge_mlp_kernel,
        grid=grid,
        in_specs=[
            pl.BlockSpec((16, _EBLK), lambda i: (0, i)),
            pl.BlockSpec((12, 4), lambda i: (0, 0)),
            wspec(12), wspec(12), wspec(12),
            pl.BlockSpec((384, 4), lambda i: (0, 0)),
            wspec(384), wspec(384), wspec(384),
        ],
        out_specs=pl.BlockSpec((384, _EBLK), lambda i: (0, i)),
        out_shape=jax.ShapeDtypeStruct((384, N_EDGES), jnp.float32),
    )(g, w1s, b1c, lw1c, lb1c, w2s, b2c, lw2c, lb2c)


def _mlp_ln_kernel(x_ref, w_ref, b_ref, lw_ref, lb_ref, o_ref):
    h = jax.nn.relu(
        lax.dot_general(x_ref[...], w_ref[...], (((1,), (0,)), ((), ())),
                        preferred_element_type=jnp.float32) + b_ref[...])
    m = jnp.mean(h, axis=-1, keepdims=True)
    v = jnp.mean((h - m) ** 2, axis=-1, keepdims=True)
    o_ref[...] = (h - m) / jnp.sqrt(v + 1e-5) * lw_ref[...] + lb_ref[...]


def _mlp_ln(x, W, b, lw, lb, blk):
    n, din = x.shape
    dout = W.shape[1]
    return pl.pallas_call(
        _mlp_ln_kernel,
        grid=(n // blk,),
        in_specs=[
            pl.BlockSpec((blk, din), lambda i: (i, 0)),
            pl.BlockSpec((din, dout), lambda i: (0, 0)),
            pl.BlockSpec((dout,), lambda i: (0,)),
            pl.BlockSpec((dout,), lambda i: (0,)),
            pl.BlockSpec((dout,), lambda i: (0,)),
        ],
        out_specs=pl.BlockSpec((blk, dout), lambda i: (i, 0)),
        out_shape=jax.ShapeDtypeStruct((n, dout), jnp.float32),
    )(x, W, b, lw, lb)


def kernel(pos_A, normal_A, pos_B, normal_B, conv_W1, conv_b1, conv_ln1_w,
           conv_ln1_b, conv_W2, conv_b2, conv_ln2_w, conv_ln2_b, Wa, ba,
           lna_w, lna_b, Wr, br, lnr_w, lnr_b, Wl, bl, edge_index_A,
           edge_index_B, residue_ids_A, residue_ids_B, src_res_idx,
           tgt_res_idx):
    f32 = jnp.float32
    w1s = jnp.transpose(conv_W1, (0, 2, 1)).reshape(12, 4)
    b1c = conv_b1.reshape(12, 1)
    lw1c = conv_ln1_w.reshape(12, 1)
    lb1c = conv_ln1_b.reshape(12, 1)
    w2s = jnp.transpose(conv_W2, (0, 2, 1)).reshape(384, 4)
    b2c = conv_b2.reshape(384, 1)
    lw2c = conv_ln2_w.reshape(384, 1)
    lb2c = conv_ln2_b.reshape(384, 1)

    def edge_stage(pos, normal, edge_index):
        table = jnp.concatenate(
            [pos, normal, jnp.zeros((N_NODES, 2), f32)], axis=1).reshape(-1)
        g = _sc_edge_gather(table, edge_index[0], edge_index[1])
        return _edge_mlp(g, w1s, b1c, lw1c, lb1c, w2s, b2c, lw2c, lb2c)

    h2tA = edge_stage(pos_A, normal_A, edge_index_A)
    h2tB = edge_stage(pos_B, normal_B, edge_index_B)
    nfA, nfB = _sc_segmax(h2tA, h2tB, edge_index_A[1], edge_index_B[1])

    def node_stage(nf, res_ids):
        atom = _mlp_ln(nf[:, :N_NODES].T, Wa, ba, lna_w, lna_b, blk=1000)
        res = jax.ops.segment_max(atom, res_ids, num_segments=N_RES)
        res = jnp.where(jnp.isneginf(res), 0.0, res)
        return _mlp_ln(res, Wr, br, lnr_w, lnr_b, blk=1000)

    res_A = node_stage(nfA, residue_ids_A)
    res_B = node_stage(nfB, residue_ids_B)
    x_s = res_A[src_res_idx]
    x_t = res_B[tgt_res_idx]
    out = jax.nn.sigmoid((x_s * x_t) @ Wl + bl)[:, 0]
    return out


# speedup vs baseline: 1.8622x; 1.0033x over previous
import functools

import jax
import jax.numpy as jnp
from jax import lax
from jax.experimental import pallas as pl
from jax.experimental.pallas import tpu as pltpu
from jax.experimental.pallas import tpu_sc as plsc

N_NODES = 10000
N_EDGES = 320000
N_RES = 1000
NC, NS, LANES = 2, 16, 16
NW = NC * NS

_GCH = 2560
_NCHUNKS = N_EDGES // _GCH


def _sc_edge_gather(table_flat, src, dst):
    mesh = plsc.VectorSubcoreMesh(core_axis_name="c", subcore_axis_name="s")

    @functools.partial(
        pl.kernel,
        out_type=jax.ShapeDtypeStruct((16, N_EDGES), jnp.float32),
        mesh=mesh,
        scratch_types=[
            pltpu.VMEM((N_NODES * 8,), jnp.float32),
            pltpu.VMEM((_GCH,), jnp.int32),
            pltpu.VMEM((_GCH,), jnp.int32),
            pltpu.VMEM((16 * _GCH,), jnp.float32),
        ],
        compiler_params=pltpu.CompilerParams(needs_layout_passes=False),
    )
    def k(tab_hbm, src_hbm, dst_hbm, out_hbm, tab_v, si_v, di_v, gb_v):
        wid = lax.axis_index("s") * NC + lax.axis_index("c")
        pltpu.sync_copy(tab_hbm, tab_v)
        nch = jnp.where(wid < _NCHUNKS - NW * (_NCHUNKS // NW),
                        _NCHUNKS // NW + 1, _NCHUNKS // NW)

        def chunk(i, carry):
            base = (wid + i * NW) * _GCH
            pltpu.sync_copy(src_hbm.at[pl.ds(base, _GCH)], si_v)
            pltpu.sync_copy(dst_hbm.at[pl.ds(base, _GCH)], di_v)

            def grp(g, c2):
                s16 = si_v[pl.ds(g * 16, 16)] * 8
                d16 = di_v[pl.ds(g * 16, 16)] * 8
                for c in range(3):
                    gb_v[pl.ds(c * _GCH + g * 16, 16)] = plsc.load_gather(
                        tab_v, [s16 + c])
                    gb_v[pl.ds((3 + c) * _GCH + g * 16, 16)] = \
                        plsc.load_gather(tab_v, [d16 + c])
                    gb_v[pl.ds((6 + c) * _GCH + g * 16, 16)] = \
                        plsc.load_gather(tab_v, [s16 + 3 + c])
                    gb_v[pl.ds((9 + c) * _GCH + g * 16, 16)] = \
                        plsc.load_gather(tab_v, [d16 + 3 + c])
                return c2

            lax.fori_loop(0, _GCH // 16, grp, 0)
            for c in range(12):
                pltpu.sync_copy(gb_v.at[pl.ds(c * _GCH, _GCH)],
                                out_hbm.at[c, pl.ds(base, _GCH)])
            return carry

        lax.fori_loop(0, nch, chunk, 0)

    return k(table_flat, src, dst)


_NPAD = 10240
_SBLK = 2560
_SNCH = N_EDGES // _SBLK


def _vperm(x, i):
    return jnp.take_along_axis(x, i, axis=0)


def _sc_segmax(h2a, h2b, dsta, dstb):
    mesh = plsc.VectorSubcoreMesh(core_axis_name="c", subcore_axis_name="s")
    f32 = jnp.float32

    @functools.partial(
        pl.kernel,
        out_type=(jax.ShapeDtypeStruct((384, _NPAD), f32),
                  jax.ShapeDtypeStruct((384, _NPAD), f32)),
        mesh=mesh,
        scratch_types=[
            [pltpu.VMEM((_NPAD,), f32) for _ in range(8)],
            pltpu.VMEM((2 * 8 * _SBLK,), f32),
            pltpu.VMEM((2 * _SBLK,), jnp.int32),
            pltpu.VMEM((2048,), jnp.int32),
            pltpu.SemaphoreType.DMA,
            pltpu.SemaphoreType.DMA,
        ],
        compiler_params=pltpu.CompilerParams(needs_layout_passes=False),
    )
    def k(h2a_hbm, h2b_hbm, da_hbm, db_hbm, oa_hbm, ob_hbm, accs, h_v, d_v,
          tag_v, sem0, sem1):
        wid = lax.axis_index("s") * NC + lax.axis_index("c")
        iota = lax.iota(jnp.int32, 16)
        ninf = jnp.full((16,), -jnp.inf, f32)
        zeros = jnp.zeros((16,), f32)

        def run(h2_hbm, d_hbm, o_hbm, row0):
            def ini(i, c):
                for f in range(8):
                    accs[f][pl.ds(i * 16, 16)] = ninf
                return c
            lax.fori_loop(0, _NPAD // 16, ini, 0)

            def fire(ch, b, sem):
                base = ch * _SBLK
                pltpu.async_copy(d_hbm.at[pl.ds(base, _SBLK)],
                                 d_v.at[pl.ds(b * _SBLK, _SBLK)], sem)
                for f in range(8):
                    pltpu.async_copy(
                        h2_hbm.at[row0 + f, pl.ds(base, _SBLK)],
                        h_v.at[pl.ds((b * 8 + f) * _SBLK, _SBLK)], sem)

            def drain(ch, b, sem):
                base = ch * _SBLK
                pltpu.make_async_copy(
                    d_hbm.at[pl.ds(base, _SBLK)],
                    d_v.at[pl.ds(b * _SBLK, _SBLK)], sem).wait()
                for f in range(8):
                    pltpu.make_async_copy(
                        h2_hbm.at[row0 + f, pl.ds(base, _SBLK)],
                        h_v.at[pl.ds((b * 8 + f) * _SBLK, _SBLK)], sem).wait()

            def process(b):
                def grp(g2, c2):
                    for u in range(2):
                        _one_group(b, g2 * 2 + u)
                    return c2

                def _one_group(b, g):
                    d16 = d_v[pl.ds(b * _SBLK + g * 16, 16)]
                    dh = d16 & 2047
                    plsc.store_scatter(tag_v, [dh], iota)
                    rb = plsc.load_gather(tag_v, [dh])
                    dup_any = jnp.any(rb != iota)
                    vals = [h_v[pl.ds((b * 8 + f) * _SBLK + g * 16, 16)]
                            for f in range(8)]

                    def fast():
                        curs = [plsc.load_gather(accs[f], [d16])
                                for f in range(8)]
                        for f in range(8):
                            plsc.store_scatter(accs[f], [d16],
                                               jnp.maximum(curs[f], vals[f]))

                    def slow():
                        ks, pm = plsc.sort_key_val(d16, iota)
                        knext = _vperm(ks, jnp.minimum(iota + 1, 15))
                        mlast = (ks != knext) | (iota == 15)
                        sames = []
                        for kk in (1, 2, 4, 8):
                            kp = _vperm(ks, jnp.maximum(iota - kk, 0))
                            sames.append((ks == kp) & (iota >= kk))
                        vps = []
                        for f in range(8):
                            vp = _vperm(vals[f], pm)
                            for kk, sm in zip((1, 2, 4, 8), sames):
                                sh = _vperm(vp, jnp.maximum(iota - kk, 0))
                                vp = jnp.where(sm, jnp.maximum(vp, sh), vp)
                            vps.append(vp)
                        curs = [plsc.load_gather(accs[f], [ks], mask=mlast)
                                for f in range(8)]
                        for f in range(8):
                            plsc.store_scatter(accs[f], [ks],
                                               jnp.maximum(curs[f], vps[f]),
                                               mask=mlast)

                    lax.cond(dup_any, slow, fast)

                lax.fori_loop(0, _SBLK // 32, grp, 0)

            fire(0, 0, sem0)

            def dbl(j, c):
                fire(2 * j + 1, 1, sem1)
                drain(2 * j, 0, sem0)
                process(0)
                fire(2 * j + 2, 0, sem0)
                drain(2 * j + 1, 1, sem1)
                process(1)
                return c

            lax.fori_loop(0, (_SNCH - 1) // 2, dbl, 0)
            drain(_SNCH - 1, 0, sem0)
            process(0)

            def fin(i, c):
                for f in range(8):
                    v = accs[f][pl.ds(i * 16, 16)]
                    accs[f][pl.ds(i * 16, 16)] = jnp.where(v == ninf, zeros, v)
                return c
            lax.fori_loop(0, _NPAD // 16, fin, 0)
            for f in range(8):
                pltpu.sync_copy(accs[f], o_hbm.at[row0 + f, :])

        for k3 in range(3):
            gid = wid + NW * k3
            lax.cond(
                gid >= 48,
                lambda gid=gid: run(h2b_hbm, db_hbm, ob_hbm, (gid - 48) * 8),
                lambda gid=gid: run(h2a_hbm, da_hbm, oa_hbm, gid * 8),
            )

    return k(h2a, h2b, dsta, dstb)


_EBLK = 2560


def _edge_mlp_kernel(g_ref, w1s_ref, b1_ref, lw1_ref, lb1_ref, w2s_ref,
                     b2_ref, lw2_ref, lb2_ref, o_ref):
    g = g_ref[...]
    d = g[0:3, :] - g[3:6, :]
    nj = g[6:9, :]
    ni = g[9:12, :]
    dn = jnp.sqrt(jnp.sum(d * d, axis=0, keepdims=True) + 1e-12)

    def ang(v1, v2):
        cx = v1[1:2, :] * v2[2:3, :] - v1[2:3, :] * v2[1:2, :]
        cy = v1[2:3, :] * v2[0:1, :] - v1[0:1, :] * v2[2:3, :]
        cz = v1[0:1, :] * v2[1:2, :] - v1[1:2, :] * v2[0:1, :]
        cn = jnp.sqrt(cx * cx + cy * cy + cz * cz + 1e-12)
        dt = jnp.sum(v1 * v2, axis=0, keepdims=True)
        return jnp.arctan2(cn, dt)

    ppf = jnp.concatenate([dn, ang(ni, d), ang(nj, d), ang(ni, nj)], axis=0)
    ones4 = jnp.ones((1, 4), jnp.float32)
    ones128 = jnp.ones((1, 128), jnp.float32)
    for i in range(3):
        w1t = w1s_ref[4 * i:4 * i + 4, :]
        p = jax.nn.relu(
            lax.dot_general(w1t, ppf, (((1,), (0,)), ((), ())),
                            preferred_element_type=jnp.float32)
            + b1_ref[4 * i:4 * i + 4, :])
        m = lax.dot_general(ones4, p, (((1,), (0,)), ((), ())),
                            preferred_element_type=jnp.float32) * 0.25
        pc = p - m
        v = lax.dot_general(ones4, pc * pc, (((1,), (0,)), ((), ())),
                            preferred_element_type=jnp.float32) * 0.25
        h1 = pc / jnp.sqrt(v + 1e-5) * lw1_ref[4 * i:4 * i + 4, :] \
            + lb1_ref[4 * i:4 * i + 4, :]
        w2t = w2s_ref[128 * i:128 * i + 128, :]
        q = jax.nn.relu(
            lax.dot_general(w2t, h1, (((1,), (0,)), ((), ())),
                            preferred_element_type=jnp.float32)
            + b2_ref[128 * i:128 * i + 128, :])
        m2 = lax.dot_general(ones128, q, (((1,), (0,)), ((), ())),
                             preferred_element_type=jnp.float32) * (1.0 / 128.0)
        qc = q - m2
        v2 = lax.dot_general(ones128, qc * qc, (((1,), (0,)), ((), ())),
                             preferred_element_type=jnp.float32) * (1.0 / 128.0)
        o_ref[128 * i:128 * i + 128, :] = (
            qc / jnp.sqrt(v2 + 1e-5) * lw2_ref[128 * i:128 * i + 128, :]
            + lb2_ref[128 * i:128 * i + 128, :])


def _edge_mlp(g, w1s, b1c, lw1c, lb1c, w2s, b2c, lw2c, lb2c):
    grid = (N_EDGES // _EBLK,)
    wspec = lambda r: pl.BlockSpec((r, 1), lambda i: (0, 0))
    return pl.pallas_call(
        _edge_mlp_kernel,
        grid=grid,
        in_specs=[
            pl.BlockSpec((16, _EBLK), lambda i: (0, i)),
            pl.BlockSpec((12, 4), lambda i: (0, 0)),
            wspec(12), wspec(12), wspec(12),
            pl.BlockSpec((384, 4), lambda i: (0, 0)),
            wspec(384), wspec(384), wspec(384),
        ],
        out_specs=pl.BlockSpec((384, _EBLK), lambda i: (0, i)),
        out_shape=jax.ShapeDtypeStruct((384, N_EDGES), jnp.float32),
    )(g, w1s, b1c, lw1c, lb1c, w2s, b2c, lw2c, lb2c)


def _mlp_ln_kernel(x_ref, w_ref, b_ref, lw_ref, lb_ref, o_ref):
    h = jax.nn.relu(
        lax.dot_general(x_ref[...], w_ref[...], (((1,), (0,)), ((), ())),
                        preferred_element_type=jnp.float32) + b_ref[...])
    m = jnp.mean(h, axis=-1, keepdims=True)
    v = jnp.mean((h - m) ** 2, axis=-1, keepdims=True)
    o_ref[...] = (h - m) / jnp.sqrt(v + 1e-5) * lw_ref[...] + lb_ref[...]


def _mlp_ln(x, W, b, lw, lb, blk):
    n, din = x.shape
    dout = W.shape[1]
    return pl.pallas_call(
        _mlp_ln_kernel,
        grid=(n // blk,),
        in_specs=[
            pl.BlockSpec((blk, din), lambda i: (i, 0)),
            pl.BlockSpec((din, dout), lambda i: (0, 0)),
            pl.BlockSpec((dout,), lambda i: (0,)),
            pl.BlockSpec((dout,), lambda i: (0,)),
            pl.BlockSpec((dout,), lambda i: (0,)),
        ],
        out_specs=pl.BlockSpec((blk, dout), lambda i: (i, 0)),
        out_shape=jax.ShapeDtypeStruct((n, dout), jnp.float32),
    )(x, W, b, lw, lb)


def kernel(pos_A, normal_A, pos_B, normal_B, conv_W1, conv_b1, conv_ln1_w,
           conv_ln1_b, conv_W2, conv_b2, conv_ln2_w, conv_ln2_b, Wa, ba,
           lna_w, lna_b, Wr, br, lnr_w, lnr_b, Wl, bl, edge_index_A,
           edge_index_B, residue_ids_A, residue_ids_B, src_res_idx,
           tgt_res_idx):
    f32 = jnp.float32
    w1s = jnp.transpose(conv_W1, (0, 2, 1)).reshape(12, 4)
    b1c = conv_b1.reshape(12, 1)
    lw1c = conv_ln1_w.reshape(12, 1)
    lb1c = conv_ln1_b.reshape(12, 1)
    w2s = jnp.transpose(conv_W2, (0, 2, 1)).reshape(384, 4)
    b2c = conv_b2.reshape(384, 1)
    lw2c = conv_ln2_w.reshape(384, 1)
    lb2c = conv_ln2_b.reshape(384, 1)

    def edge_stage(pos, normal, edge_index):
        table = jnp.concatenate(
            [pos, normal, jnp.zeros((N_NODES, 2), f32)], axis=1).reshape(-1)
        g = _sc_edge_gather(table, edge_index[0], edge_index[1])
        return _edge_mlp(g, w1s, b1c, lw1c, lb1c, w2s, b2c, lw2c, lb2c)

    h2tA = edge_stage(pos_A, normal_A, edge_index_A)
    h2tB = edge_stage(pos_B, normal_B, edge_index_B)
    nfA, nfB = _sc_segmax(h2tA, h2tB, edge_index_A[1], edge_index_B[1])

    def node_stage(nf, res_ids):
        atom = _mlp_ln(nf[:, :N_NODES].T, Wa, ba, lna_w, lna_b, blk=1000)
        res = jax.ops.segment_max(atom, res_ids, num_segments=N_RES)
        res = jnp.where(jnp.isneginf(res), 0.0, res)
        return _mlp_ln(res, Wr, br, lnr_w, lnr_b, blk=1000)

    res_A = node_stage(nfA, residue_ids_A)
    res_B = node_stage(nfB, residue_ids_B)
    x_s = res_A[src_res_idx]
    x_t = res_B[tgt_res_idx]
    out = jax.nn.sigmoid((x_s * x_t) @ Wl + bl)[:, 0]
    return out
